# baseline (speedup 1.0000x reference)
import dataclasses

import jax
import jax.numpy as jnp
from jax import lax
from jax.experimental import pallas as pl
from jax.experimental.pallas import tpu as pltpu
from jax.experimental.pallas import tpu_sc as plsc

_N = 10000
_E = 320000
_NFEAT = 128
_NHID = 128
_NCLASS = 64

_NC = 2
_NS = 16
_NW = _NC * _NS
_LANES = 16

_CHUNK = 112
_NBUF = 3
_NPK = 6
_CPW0 = 60
_CPW1 = 120
_NCH = _NS * (_CPW0 + _CPW1)
_E_PAD = _NCH * _CHUNK
_RPS = 632
_NA = _NS * _RPS
_RPS_LAST = _N - 15 * _RPS

_ROWS_N = _N // 10


def _support_body(x_ref, w_ref, o_ref):
    o_ref[...] = jnp.dot(x_ref[...], w_ref[...],
                         preferred_element_type=jnp.float32)


_support_mm = pl.pallas_call(
    _support_body,
    grid=(10,),
    in_specs=[
        pl.BlockSpec((_ROWS_N, _NFEAT), lambda i: (i, 0)),
        pl.BlockSpec((_NFEAT, _NHID), lambda i: (0, 0)),
    ],
    out_specs=pl.BlockSpec((_ROWS_N, _NHID), lambda i: (i, 0)),
    out_shape=jax.ShapeDtypeStruct((_N, _NHID), jnp.float32),
)


def _head_body(p0_ref, p1_ref, bgc_ref, wh_ref, bh_ref, feat_ref, out_ref):
    feat = jnp.maximum(p0_ref[...] + p1_ref[...] + bgc_ref[...], 0.0)
    feat_ref[...] = feat
    logits = jnp.dot(feat, wh_ref[...], preferred_element_type=jnp.float32)
    out_ref[...] = jax.nn.sigmoid(logits + bh_ref[...])


_head = pl.pallas_call(
    _head_body,
    grid=(10,),
    in_specs=[
        pl.BlockSpec((_ROWS_N, _NHID), lambda i: (i, 0)),
        pl.BlockSpec((_ROWS_N, _NHID), lambda i: (i, 0)),
        pl.BlockSpec((_NHID,), lambda i: (0,)),
        pl.BlockSpec((_NHID, _NCLASS), lambda i: (0, 0)),
        pl.BlockSpec((_NCLASS,), lambda i: (0,)),
    ],
    out_specs=[
        pl.BlockSpec((_ROWS_N, _NHID), lambda i: (i, 0)),
        pl.BlockSpec((_ROWS_N, _NCLASS), lambda i: (i, 0)),
    ],
    out_shape=[
        jax.ShapeDtypeStruct((_N, _NHID), jnp.float32),
        jax.ShapeDtypeStruct((_N, _NCLASS), jnp.float32),
    ],
)


def _scale_rows(rows, pk):
    @pl.loop(0, _CHUNK // _LANES)
    def _(g):
        vals16 = plsc.bitcast(pk[2, pl.ds(g * _LANES, _LANES)],
                              jnp.float32)
        for i in range(_LANES):
            v = vals16[i]
            r = g * _LANES + i
            for j in range(_NHID // _LANES):
                sl = (r, pl.ds(j * _LANES, _LANES))
                rows[sl] = rows[sl] * v


def _run_edges(support_hbm, pk_hbm, shared, pkv, rowsv,
               sem_g, sem_i, wchunk, cpw):
    for r in range(_NBUF):
        pltpu.sync_copy(pk_hbm.at[wchunk + r], pkv[r])
        pltpu.async_copy(support_hbm.at[pkv[r].at[0]], rowsv[r], sem_g)
    for r in range(_NBUF, _NPK):
        pltpu.async_copy(pk_hbm.at[wchunk + r], pkv[r], sem_i)

    @pl.loop(0, cpw, step=6)
    def _(k):
        for b in range(6):
            kk = k + b
            pk = pkv[b % _NPK]
            rows = rowsv[b % _NBUF]

            pltpu.make_async_copy(
                support_hbm.at[pk.at[0]], rows, sem_g).wait()

            _scale_rows(rows, pk)
            pltpu.sync_copy(rows, shared.at[pk.at[1]], add=True)

            @pl.when(kk + _NBUF < cpw)
            def _():
                pkn = pkv[(b + _NBUF) % _NPK]
                pltpu.make_async_copy(
                    pk_hbm.at[wchunk + kk + _NBUF], pkn, sem_i).wait()
                pltpu.async_copy(support_hbm.at[pkn.at[0]], rows, sem_g)

            @pl.when(kk + _NPK < cpw)
            def _():
                pltpu.async_copy(pk_hbm.at[wchunk + kk + _NPK],
                                 pkv[b % _NPK], sem_i)


def _sc_body(support_hbm, pk_hbm, out_hbm,
             pk0, pk1, pk2, pk3, pk4, pk5,
             rows0, rows1, rows2, shared, sem_g, sem_i):
    c = lax.axis_index("c")
    s = lax.axis_index("s")

    pkv = (pk0, pk1, pk2, pk3, pk4, pk5)
    rowsv = (rows0, rows1, rows2)

    @pl.loop(0, _CHUNK)
    def _(r):
        for j in range(_NHID // _LANES):
            rows0[r, pl.ds(j * _LANES, _LANES)] = jnp.zeros(
                (_LANES,), jnp.float32)

    for t in range(5):
        pltpu.sync_copy(rows0,
                        shared.at[pl.ds(s * _RPS + t * _CHUNK, _CHUNK)])
    pltpu.sync_copy(rows0.at[pl.ds(0, _RPS - 5 * _CHUNK)],
                    shared.at[pl.ds(s * _RPS + 5 * _CHUNK,
                                    _RPS - 5 * _CHUNK)])

    plsc.subcore_barrier()

    @pl.when(c == 0)
    def _():
        _run_edges(support_hbm, pk_hbm, shared, pkv, rowsv,
                   sem_g, sem_i, s * _CPW0, _CPW0)

    @pl.when(c == 1)
    def _():
        _run_edges(support_hbm, pk_hbm, shared, pkv, rowsv,
                   sem_g, sem_i, _NS * _CPW0 + s * _CPW1, _CPW1)

    plsc.subcore_barrier()

    @pl.when(s < _NS - 1)
    def _():
        pltpu.sync_copy(shared.at[pl.ds(s * _RPS, _RPS)],
                        out_hbm.at[c].at[pl.ds(s * _RPS, _RPS)])

    @pl.when(s == _NS - 1)
    def _():
        pltpu.sync_copy(shared.at[pl.ds((_NS - 1) * _RPS, _RPS_LAST)],
                        out_hbm.at[c].at[pl.ds((_NS - 1) * _RPS, _RPS_LAST)])


_sc_params = pltpu.CompilerParams()
if "needs_layout_passes" in pltpu.CompilerParams.__dataclass_fields__:
    _sc_params = dataclasses.replace(_sc_params, needs_layout_passes=False)

_sc_spmm = pl.kernel(
    _sc_body,
    out_type=jax.ShapeDtypeStruct((_NC, _N, _NHID), jnp.float32),
    mesh=plsc.VectorSubcoreMesh(core_axis_name="c", subcore_axis_name="s"),
    compiler_params=_sc_params,
    scratch_types=(
        [pltpu.VMEM((3, _CHUNK), jnp.int32) for _ in range(_NPK)]
        + [pltpu.VMEM((_CHUNK, _NHID), jnp.float32) for _ in range(_NBUF)]
        + [
            pltpu.VMEM_SHARED((_NA, _NHID), jnp.float32),
            pltpu.SemaphoreType.DMA,
            pltpu.SemaphoreType.DMA,
        ]
    ),
)


def kernel(x, adj_indices, adj_values, W_gc, b_gc, W_hash, b_hash):
    support = _support_mm(x, W_gc)

    pad = _E_PAD - _E
    src = jnp.pad(adj_indices[0], (0, pad))
    dst = jnp.pad(adj_indices[1], (0, pad))
    vbits = jax.lax.bitcast_convert_type(
        jnp.pad(adj_values, (0, pad)), jnp.int32)
    pk = jnp.stack([src, dst, vbits], axis=0)
    pk = pk.reshape(3, _NCH, _CHUNK).transpose(1, 0, 2)

    partials = _sc_spmm(support, pk)
    feature, out = _head(partials[0], partials[1], b_gc, W_hash, b_hash)
    return (feature, out)


# speedup vs baseline: 1.1800x; 1.1800x over previous
import dataclasses

import jax
import jax.numpy as jnp
from jax import lax
from jax.experimental import pallas as pl
from jax.experimental.pallas import tpu as pltpu
from jax.experimental.pallas import tpu_sc as plsc

_N = 10000
_E = 320000
_NFEAT = 128
_NHID = 128
_NCLASS = 64

_NC = 2
_NS = 16
_NW = _NC * _NS
_LANES = 16

_CHUNK = 112
_NBUF = 3
_NPK = 6
_CPW0 = 120
_CPW1 = 60
_NCH = _NS * (_CPW0 + _CPW1)
_E_PAD = _NCH * _CHUNK
_RPS = 632
_NA = _NS * _RPS
_RPS_LAST = _N - 15 * _RPS

_ROWS_N = _N // 10


def _support_body(x_ref, w_ref, o_ref):
    o_ref[...] = jnp.dot(x_ref[...], w_ref[...],
                         preferred_element_type=jnp.float32)


_support_mm = pl.pallas_call(
    _support_body,
    grid=(10,),
    in_specs=[
        pl.BlockSpec((_ROWS_N, _NFEAT), lambda i: (i, 0)),
        pl.BlockSpec((_NFEAT, _NHID), lambda i: (0, 0)),
    ],
    out_specs=pl.BlockSpec((_ROWS_N, _NHID), lambda i: (i, 0)),
    out_shape=jax.ShapeDtypeStruct((_N, _NHID), jnp.float32),
)


def _head_body(p0_ref, p1_ref, bgc_ref, wh_ref, bh_ref, feat_ref, out_ref):
    feat = jnp.maximum(p0_ref[...] + p1_ref[...] + bgc_ref[...], 0.0)
    feat_ref[...] = feat
    logits = jnp.dot(feat, wh_ref[...], preferred_element_type=jnp.float32)
    out_ref[...] = jax.nn.sigmoid(logits + bh_ref[...])


_head = pl.pallas_call(
    _head_body,
    grid=(10,),
    in_specs=[
        pl.BlockSpec((_ROWS_N, _NHID), lambda i: (i, 0)),
        pl.BlockSpec((_ROWS_N, _NHID), lambda i: (i, 0)),
        pl.BlockSpec((_NHID,), lambda i: (0,)),
        pl.BlockSpec((_NHID, _NCLASS), lambda i: (0, 0)),
        pl.BlockSpec((_NCLASS,), lambda i: (0,)),
    ],
    out_specs=[
        pl.BlockSpec((_ROWS_N, _NHID), lambda i: (i, 0)),
        pl.BlockSpec((_ROWS_N, _NCLASS), lambda i: (i, 0)),
    ],
    out_shape=[
        jax.ShapeDtypeStruct((_N, _NHID), jnp.float32),
        jax.ShapeDtypeStruct((_N, _NCLASS), jnp.float32),
    ],
)


def _scale_rows(rows, pk):
    @pl.loop(0, _CHUNK // _LANES)
    def _(g):
        vals16 = plsc.bitcast(pk[2, pl.ds(g * _LANES, _LANES)],
                              jnp.float32)
        for i in range(_LANES):
            v = vals16[i]
            r = g * _LANES + i
            for j in range(_NHID // _LANES):
                sl = (r, pl.ds(j * _LANES, _LANES))
                rows[sl] = rows[sl] * v


def _run_edges(support_hbm, pk_hbm, shared, pkv, rowsv,
               sem_g, sem_i, wchunk, cpw):
    for r in range(_NBUF):
        pltpu.sync_copy(pk_hbm.at[wchunk + r], pkv[r])
        pltpu.async_copy(support_hbm.at[pkv[r].at[0]], rowsv[r], sem_g)
    for r in range(_NBUF, _NPK):
        pltpu.async_copy(pk_hbm.at[wchunk + r], pkv[r], sem_i)

    @pl.loop(0, cpw, step=6)
    def _(k):
        for b in range(6):
            kk = k + b
            pk = pkv[b % _NPK]
            rows = rowsv[b % _NBUF]

            pltpu.make_async_copy(
                support_hbm.at[pk.at[0]], rows, sem_g).wait()

            _scale_rows(rows, pk)
            pltpu.sync_copy(rows, shared.at[pk.at[1]], add=True)

            @pl.when(kk + _NBUF < cpw)
            def _():
                pkn = pkv[(b + _NBUF) % _NPK]
                pltpu.make_async_copy(
                    pk_hbm.at[wchunk + kk + _NBUF], pkn, sem_i).wait()
                pltpu.async_copy(support_hbm.at[pkn.at[0]], rows, sem_g)

            @pl.when(kk + _NPK < cpw)
            def _():
                pltpu.async_copy(pk_hbm.at[wchunk + kk + _NPK],
                                 pkv[b % _NPK], sem_i)


def _sc_body(support_hbm, pk_hbm, out_hbm,
             pk0, pk1, pk2, pk3, pk4, pk5,
             rows0, rows1, rows2, shared, sem_g, sem_i):
    c = lax.axis_index("c")
    s = lax.axis_index("s")

    pkv = (pk0, pk1, pk2, pk3, pk4, pk5)
    rowsv = (rows0, rows1, rows2)

    @pl.loop(0, _CHUNK)
    def _(r):
        for j in range(_NHID // _LANES):
            rows0[r, pl.ds(j * _LANES, _LANES)] = jnp.zeros(
                (_LANES,), jnp.float32)

    for t in range(5):
        pltpu.sync_copy(rows0,
                        shared.at[pl.ds(s * _RPS + t * _CHUNK, _CHUNK)])
    pltpu.sync_copy(rows0.at[pl.ds(0, _RPS - 5 * _CHUNK)],
                    shared.at[pl.ds(s * _RPS + 5 * _CHUNK,
                                    _RPS - 5 * _CHUNK)])

    plsc.subcore_barrier()

    @pl.when(c == 0)
    def _():
        _run_edges(support_hbm, pk_hbm, shared, pkv, rowsv,
                   sem_g, sem_i, s * _CPW0, _CPW0)

    @pl.when(c == 1)
    def _():
        _run_edges(support_hbm, pk_hbm, shared, pkv, rowsv,
                   sem_g, sem_i, _NS * _CPW0 + s * _CPW1, _CPW1)

    plsc.subcore_barrier()

    @pl.when(s < _NS - 1)
    def _():
        pltpu.sync_copy(shared.at[pl.ds(s * _RPS, _RPS)],
                        out_hbm.at[c].at[pl.ds(s * _RPS, _RPS)])

    @pl.when(s == _NS - 1)
    def _():
        pltpu.sync_copy(shared.at[pl.ds((_NS - 1) * _RPS, _RPS_LAST)],
                        out_hbm.at[c].at[pl.ds((_NS - 1) * _RPS, _RPS_LAST)])


_sc_params = pltpu.CompilerParams()
if "needs_layout_passes" in pltpu.CompilerParams.__dataclass_fields__:
    _sc_params = dataclasses.replace(_sc_params, needs_layout_passes=False)

_sc_spmm = pl.kernel(
    _sc_body,
    out_type=jax.ShapeDtypeStruct((_NC, _N, _NHID), jnp.float32),
    mesh=plsc.VectorSubcoreMesh(core_axis_name="c", subcore_axis_name="s"),
    compiler_params=_sc_params,
    scratch_types=(
        [pltpu.VMEM((3, _CHUNK), jnp.int32) for _ in range(_NPK)]
        + [pltpu.VMEM((_CHUNK, _NHID), jnp.float32) for _ in range(_NBUF)]
        + [
            pltpu.VMEM_SHARED((_NA, _NHID), jnp.float32),
            pltpu.SemaphoreType.DMA,
            pltpu.SemaphoreType.DMA,
        ]
    ),
)


def kernel(x, adj_indices, adj_values, W_gc, b_gc, W_hash, b_hash):
    support = _support_mm(x, W_gc)

    pad = _E_PAD - _E
    src = jnp.pad(adj_indices[0], (0, pad))
    dst = jnp.pad(adj_indices[1], (0, pad))
    vbits = jax.lax.bitcast_convert_type(
        jnp.pad(adj_values, (0, pad)), jnp.int32)
    pk = jnp.stack([src, dst, vbits], axis=0)
    pk = pk.reshape(3, _NCH, _CHUNK).transpose(1, 0, 2)

    partials = _sc_spmm(support, pk)
    feature, out = _head(partials[0], partials[1], b_gc, W_hash, b_hash)
    return (feature, out)
